# baseline (speedup 1.0000x reference)
import functools

import jax
import jax.numpy as jnp
from jax import lax
from jax.experimental import pallas as pl
from jax.experimental.pallas import tpu as pltpu
from jax.experimental.pallas import tpu_sc as plsc

NC, NS, L = 2, 16, 16
POSN = 64


def _make_sc_sumsq(B, T, D, R=32, unroll=4, interpret=False):
  NW = NC * NS
  assert B % NW == 0 and T % R == 0 and R % L == 0 and D % L == 0
  BPW = B // NW
  NCH = T // R
  assert NCH % 2 == 0
  G = R // L

  mesh = plsc.VectorSubcoreMesh(
      core_axis_name="c", subcore_axis_name="s",
      num_cores=NC, num_subcores=NS)

  @functools.partial(
      pl.kernel,
      out_type=jax.ShapeDtypeStruct((B, T), jnp.float32),
      mesh=mesh,
      scratch_types=[
          pltpu.VMEM((2, R, D), jnp.float32),
          pltpu.VMEM((2, R, D), jnp.float32),
          pltpu.VMEM((2, R, D), jnp.float32),
          pltpu.VMEM((D,), jnp.float32),
          pltpu.VMEM((D,), jnp.float32),
          pltpu.VMEM((D,), jnp.float32),
          pltpu.VMEM((T,), jnp.float32),
          pltpu.SemaphoreType.DMA,
          pltpu.SemaphoreType.DMA,
      ],
      compiler_params=pltpu.CompilerParams(
          use_tc_tiling_on_sc=False, needs_layout_passes=False),
      interpret=interpret,
  )
  def sc_fn(h_hbm, t_hbm, r_hbm, htr_hbm, ttr_hbm, rtr_hbm, out_hbm,
            hbuf, tbuf, rbuf, htr_v, ttr_v, rtr_v, out_v, sem0, sem1):
    wid = lax.axis_index("s") * NC + lax.axis_index("c")
    b0 = wid * BPW

    def start_chunk(b, c, slot, sem):
      src = pl.ds(c * R, R)
      pltpu.async_copy(h_hbm.at[b, src], hbuf.at[slot], sem)
      pltpu.async_copy(t_hbm.at[b, src], tbuf.at[slot], sem)
      pltpu.async_copy(r_hbm.at[b, src], rbuf.at[slot], sem)

    def wait_chunk(b, c, slot, sem):
      src = pl.ds(c * R, R)
      pltpu.make_async_copy(h_hbm.at[b, src], hbuf.at[slot], sem).wait()
      pltpu.make_async_copy(t_hbm.at[b, src], tbuf.at[slot], sem).wait()
      pltpu.make_async_copy(r_hbm.at[b, src], rbuf.at[slot], sem).wait()

    lanes = lax.iota(jnp.int32, L)
    row_idx = [jnp.int32(g * L) + lanes for g in range(G)]

    def compute_chunk(slot, c, s_rtr):
      slot_v = jnp.full((L,), slot, jnp.int32)

      def dcstep(dc, carry):
        accs = [list(a) for a in carry]
        base = dc * L
        htr_c = htr_v[pl.ds(base, L)]
        ttr_c = ttr_v[pl.ds(base, L)]
        rtr_c = rtr_v[pl.ds(base, L)]
        d_vec0 = jnp.full((L,), base, jnp.int32)
        for k in range(L):
          d_vec = d_vec0 + k
          htr_d = htr_c[k]
          ttr_d = ttr_c[k]
          rtr_d = rtr_c[k]
          for g in range(G):
            a_u2, a_ur, a_h, a_t = accs[g]
            hd = plsc.load_gather(hbuf, [slot_v, row_idx[g], d_vec])
            td = plsc.load_gather(tbuf, [slot_v, row_idx[g], d_vec])
            rd = plsc.load_gather(rbuf, [slot_v, row_idx[g], d_vec])
            u = hd + rd - td
            accs[g] = (a_u2 + u * u, a_ur + u * rtr_d,
                       a_h + hd * htr_d, a_t + td * ttr_d)
        return tuple(tuple(a) for a in accs)

      zeros = jnp.zeros((L,), jnp.float32)
      init = tuple((zeros, zeros, zeros, zeros) for _ in range(G))
      accs = lax.fori_loop(0, D // L, dcstep, init)
      for g in range(G):
        a_u2, a_ur, a_h, a_t = accs[g]
        alpha = a_h - a_t
        ss = a_u2 + 2.0 * alpha * a_ur + alpha * alpha * s_rtr
        out_v[pl.ds(c * R + g * L, L)] = ss

    def batch_body(ib, _):
      b = b0 + ib
      pltpu.sync_copy(htr_hbm.at[b], htr_v)
      pltpu.sync_copy(ttr_hbm.at[b], ttr_v)
      pltpu.sync_copy(rtr_hbm.at[b], rtr_v)
      start_chunk(b, 0, 0, sem0)
      acc = jnp.zeros((L,), jnp.float32)
      for i in range(D // L):
        v = rtr_v[pl.ds(i * L, L)]
        acc = acc + v * v
      s_rtr = acc[0]
      for k in range(1, L):
        s_rtr = s_rtr + acc[k]

      def pair_body(p, _):
        c0 = 2 * p
        start_chunk(b, c0 + 1, 1, sem1)
        wait_chunk(b, c0, 0, sem0)
        compute_chunk(0, c0, s_rtr)

        @pl.when(p + 1 < NCH // 2)
        def _():
          start_chunk(b, c0 + 2, 0, sem0)
        wait_chunk(b, c0 + 1, 1, sem1)
        compute_chunk(1, c0 + 1, s_rtr)
        return _

      lax.fori_loop(0, NCH // 2, pair_body, None)
      pltpu.sync_copy(out_v, out_hbm.at[b])
      return _

    lax.fori_loop(0, BPW, batch_body, None)

  return sc_fn


def _make_tc_negsqrt(B, T, interpret=False):
  def body(ss_ref, o_ref):
    o_ref[...] = -jnp.sqrt(ss_ref[...])

  return pl.pallas_call(
      body,
      out_shape=jax.ShapeDtypeStruct((B, T), jnp.float32),
      interpret=interpret,
  )


def kernel(h, t, r, pos_num, h_transfer, r_transfer, t_transfer):
  B, T, _, D = h.shape
  h3 = h.reshape(B, T, D)
  t3 = t.reshape(B, T, D)
  r3 = r.reshape(B, T, D)
  htr = h_transfer.reshape(B, D)
  ttr = t_transfer.reshape(B, D)
  rtr = r_transfer.reshape(B, D)
  ss = _make_sc_sumsq(B, T, D)(h3, t3, r3, htr, ttr, rtr)
  score = _make_tc_negsqrt(B, T)(ss)
  p_score = score[:, :POSN]
  n_score = lax.dynamic_slice_in_dim(score, pos_num, T - POSN, axis=1)
  return (p_score, n_score)


# speedup vs baseline: 1.1453x; 1.1453x over previous
import functools

import jax
import jax.numpy as jnp
from jax import lax
from jax.experimental import pallas as pl
from jax.experimental.pallas import tpu as pltpu
from jax.experimental.pallas import tpu_sc as plsc

NC, NS, L = 2, 16, 16
POSN = 64


def _make_sc_sumsq(B, T, D, R=32, unroll=4, interpret=False):
  NW = NC * NS
  assert B % NW == 0 and T % R == 0 and R % L == 0 and D % L == 0
  BPW = B // NW
  NCH = T // R
  assert NCH % 2 == 0
  G = R // L

  mesh = plsc.VectorSubcoreMesh(
      core_axis_name="c", subcore_axis_name="s",
      num_cores=NC, num_subcores=NS)

  @functools.partial(
      pl.kernel,
      out_type=jax.ShapeDtypeStruct((B, T), jnp.float32),
      mesh=mesh,
      scratch_types=[
          pltpu.VMEM((2, R, D), jnp.float32),
          pltpu.VMEM((2, R, D), jnp.float32),
          pltpu.VMEM((2, R, D), jnp.float32),
          pltpu.VMEM((D,), jnp.float32),
          pltpu.VMEM((D,), jnp.float32),
          pltpu.VMEM((D,), jnp.float32),
          pltpu.VMEM((T,), jnp.float32),
          pltpu.SemaphoreType.DMA,
          pltpu.SemaphoreType.DMA,
      ],
      compiler_params=pltpu.CompilerParams(
          use_tc_tiling_on_sc=False, needs_layout_passes=False),
      interpret=interpret,
  )
  def sc_fn(h_hbm, t_hbm, r_hbm, htr_hbm, ttr_hbm, rtr_hbm, out_hbm,
            hbuf, tbuf, rbuf, htr_v, ttr_v, rtr_v, out_v, sem0, sem1):
    wid = lax.axis_index("s") * NC + lax.axis_index("c")
    b0 = wid * BPW

    def start_chunk(b, c, slot, sem):
      src = pl.ds(c * R, R)
      pltpu.async_copy(h_hbm.at[b, src], hbuf.at[slot], sem)
      pltpu.async_copy(t_hbm.at[b, src], tbuf.at[slot], sem)
      pltpu.async_copy(r_hbm.at[b, src], rbuf.at[slot], sem)

    def wait_chunk(b, c, slot, sem):
      src = pl.ds(c * R, R)
      pltpu.make_async_copy(h_hbm.at[b, src], hbuf.at[slot], sem).wait()
      pltpu.make_async_copy(t_hbm.at[b, src], tbuf.at[slot], sem).wait()
      pltpu.make_async_copy(r_hbm.at[b, src], rbuf.at[slot], sem).wait()

    lanes = lax.iota(jnp.int32, L)
    row_idx = [jnp.int32(g * L) + lanes for g in range(G)]

    def compute_chunk(slot, c, s_rtr):
      slot_v = jnp.full((L,), slot, jnp.int32)

      def dcstep(dc, carry):
        accs = [list(a) for a in carry]
        base = dc * L
        htr_c = htr_v[pl.ds(base, L)]
        ttr_c = ttr_v[pl.ds(base, L)]
        rtr_c = rtr_v[pl.ds(base, L)]
        d_vec0 = jnp.full((L,), base, jnp.int32)
        for k in range(L):
          d_vec = d_vec0 + k
          htr_d = htr_c[k]
          ttr_d = ttr_c[k]
          rtr_d = rtr_c[k]
          for g in range(G):
            a_u2, a_ur, a_h, a_t = accs[g]
            hd = plsc.load_gather(hbuf, [slot_v, row_idx[g], d_vec])
            td = plsc.load_gather(tbuf, [slot_v, row_idx[g], d_vec])
            rd = plsc.load_gather(rbuf, [slot_v, row_idx[g], d_vec])
            u = hd + rd - td
            accs[g] = (a_u2 + u * u, a_ur + u * rtr_d,
                       a_h + hd * htr_d, a_t + td * ttr_d)
        return tuple(tuple(a) for a in accs)

      zeros = jnp.zeros((L,), jnp.float32)
      init = tuple((zeros, zeros, zeros, zeros) for _ in range(G))
      accs = plsc.parallel_loop(0, D // L, 1, unroll=2, carry=init)(
          lambda i, c: dcstep(i, c))
      for g in range(G):
        a_u2, a_ur, a_h, a_t = accs[g]
        alpha = a_h - a_t
        ss = a_u2 + 2.0 * alpha * a_ur + alpha * alpha * s_rtr
        out_v[pl.ds(c * R + g * L, L)] = ss

    def batch_body(ib, _):
      b = b0 + ib
      pltpu.sync_copy(htr_hbm.at[b], htr_v)
      pltpu.sync_copy(ttr_hbm.at[b], ttr_v)
      pltpu.sync_copy(rtr_hbm.at[b], rtr_v)
      start_chunk(b, 0, 0, sem0)
      acc = jnp.zeros((L,), jnp.float32)
      for i in range(D // L):
        v = rtr_v[pl.ds(i * L, L)]
        acc = acc + v * v
      s_rtr = acc[0]
      for k in range(1, L):
        s_rtr = s_rtr + acc[k]

      def pair_body(p, _):
        c0 = 2 * p
        start_chunk(b, c0 + 1, 1, sem1)
        wait_chunk(b, c0, 0, sem0)
        compute_chunk(0, c0, s_rtr)

        @pl.when(p + 1 < NCH // 2)
        def _():
          start_chunk(b, c0 + 2, 0, sem0)
        wait_chunk(b, c0 + 1, 1, sem1)
        compute_chunk(1, c0 + 1, s_rtr)
        return _

      lax.fori_loop(0, NCH // 2, pair_body, None)
      pltpu.sync_copy(out_v, out_hbm.at[b])
      return _

    lax.fori_loop(0, BPW, batch_body, None)

  return sc_fn


def _make_tc_negsqrt(B, T, interpret=False):
  def body(ss_ref, o_ref):
    o_ref[...] = -jnp.sqrt(ss_ref[...])

  return pl.pallas_call(
      body,
      out_shape=jax.ShapeDtypeStruct((B, T), jnp.float32),
      interpret=interpret,
  )


def kernel(h, t, r, pos_num, h_transfer, r_transfer, t_transfer):
  B, T, _, D = h.shape
  h3 = h.reshape(B, T, D)
  t3 = t.reshape(B, T, D)
  r3 = r.reshape(B, T, D)
  htr = h_transfer.reshape(B, D)
  ttr = t_transfer.reshape(B, D)
  rtr = r_transfer.reshape(B, D)
  ss = _make_sc_sumsq(B, T, D)(h3, t3, r3, htr, ttr, rtr)
  score = _make_tc_negsqrt(B, T)(ss)
  p_score = score[:, :POSN]
  n_score = lax.dynamic_slice_in_dim(score, pos_num, T - POSN, axis=1)
  return (p_score, n_score)
